# Initial kernel scaffold; baseline (speedup 1.0000x reference)
import jax
import jax.numpy as jnp
from jax import lax
from jax.experimental import pallas as pl
from jax.experimental.pallas import tpu as pltpu
from jax.experimental.pallas import tpu_sc as plsc

N_VARS = 100000
N_CLAUSES = 50000
L = 16
N_TILES = 32
VARS_PAD = 100352
CLAUSES_PAD = 50176
VARS_PER_TILE = VARS_PAD // N_TILES
CLAUSES_PER_TILE = CLAUSES_PAD // N_TILES
N_CHUNKS = CLAUSES_PER_TILE // L


def _sigmoid(x):
    return 1.0 / (1.0 + jnp.exp(-x))


def _sat_body(logits_hbm, vars_hbm, signs_hbm,
              assign_hbm, sats_hbm, min_hbm, cnt_hbm,
              table_v, vars_v, signs_v, sat_v, sig_v, min_v, cnt_v):
    c = lax.axis_index("c")
    s = lax.axis_index("s")
    wid = c * 16 + s

    pltpu.sync_copy(logits_hbm, table_v)

    cbase = wid * CLAUSES_PER_TILE
    for j in range(3):
        pltpu.sync_copy(vars_hbm.at[j, pl.ds(cbase, CLAUSES_PER_TILE)],
                        vars_v.at[j])
        pltpu.sync_copy(signs_hbm.at[j, pl.ds(cbase, CLAUSES_PER_TILE)],
                        signs_v.at[j])

    vbase = wid * VARS_PER_TILE

    def sig_body(i, carry):
        x = table_v[pl.ds(vbase + i * L, L)]
        sig_v[pl.ds(i * L, L)] = _sigmoid(x)
        return carry

    lax.fori_loop(0, VARS_PER_TILE // L, sig_body, 0)
    pltpu.sync_copy(sig_v, assign_hbm.at[pl.ds(vbase, VARS_PER_TILE)])

    lanes = lax.iota(jnp.int32, L)

    def chunk_body(k, carry):
        mn, ct = carry
        col = k * L
        m = None
        for j in range(3):
            idx = vars_v[j, pl.ds(col, L)]
            x = plsc.load_gather(table_v, [idx])
            sg = signs_v[j, pl.ds(col, L)]
            lit = jnp.where(sg == 1, x, -x)
            m = lit if m is None else jnp.maximum(m, lit)
        sat = _sigmoid(m)
        sat_v[pl.ds(col, L)] = sat
        valid = (cbase + col + lanes) < N_CLAUSES
        mn = jnp.minimum(mn, jnp.where(valid, sat, jnp.float32(2.0)))
        ct = ct + jnp.where(valid & (sat > 0.5),
                            jnp.float32(1.0), jnp.float32(0.0))
        return (mn, ct)

    mn0 = jnp.full((L,), 2.0, jnp.float32)
    ct0 = jnp.zeros((L,), jnp.float32)
    mn, ct = lax.fori_loop(0, N_CHUNKS, chunk_body, (mn0, ct0))

    pltpu.sync_copy(sat_v, sats_hbm.at[pl.ds(cbase, CLAUSES_PER_TILE)])
    min_v[...] = mn
    cnt_v[...] = ct
    pltpu.sync_copy(min_v, min_hbm.at[wid])
    pltpu.sync_copy(cnt_v, cnt_hbm.at[wid])


_sat_call = pl.kernel(
    _sat_body,
    out_type=[
        jax.ShapeDtypeStruct((VARS_PAD,), jnp.float32),
        jax.ShapeDtypeStruct((CLAUSES_PAD,), jnp.float32),
        jax.ShapeDtypeStruct((N_TILES, L), jnp.float32),
        jax.ShapeDtypeStruct((N_TILES, L), jnp.float32),
    ],
    mesh=plsc.VectorSubcoreMesh(core_axis_name="c", subcore_axis_name="s"),
    scratch_types=[
        pltpu.VMEM((VARS_PAD,), jnp.float32),
        pltpu.VMEM((3, CLAUSES_PER_TILE), jnp.int32),
        pltpu.VMEM((3, CLAUSES_PER_TILE), jnp.int32),
        pltpu.VMEM((CLAUSES_PER_TILE,), jnp.float32),
        pltpu.VMEM((VARS_PER_TILE,), jnp.float32),
        pltpu.VMEM((L,), jnp.float32),
        pltpu.VMEM((L,), jnp.float32),
    ],
)


@jax.jit
def kernel(assignment_logits, clause_vars, clause_signs):
    logits_p = jnp.pad(assignment_logits, (0, VARS_PAD - N_VARS))
    vars_t = jnp.pad(clause_vars.astype(jnp.int32),
                     ((0, CLAUSES_PAD - N_CLAUSES), (0, 0))).T
    signs_t = jnp.pad(clause_signs.astype(jnp.int32),
                      ((0, CLAUSES_PAD - N_CLAUSES), (0, 0))).T
    assign_p, sats_p, mins, cnts = _sat_call(logits_p, vars_t, signs_t)
    assignments = assign_p[:N_VARS]
    clause_satisfactions = sats_p[:N_CLAUSES]
    all_satisfied = jnp.min(mins)
    n_satisfied = jnp.sum(cnts)
    return (assignments, clause_satisfactions, all_satisfied, n_satisfied)


# speedup vs baseline: 2.3992x; 2.3992x over previous
import jax
import jax.numpy as jnp
from jax import lax
from jax.experimental import pallas as pl
from jax.experimental.pallas import tpu as pltpu
from jax.experimental.pallas import tpu_sc as plsc

N_VARS = 100000
N_CLAUSES = 50000
L = 16
N_TILES = 32
VARS_PAD = 100352
CLAUSES_PAD = 50176
VARS_PER_TILE = VARS_PAD // N_TILES
CLAUSES_PER_TILE = CLAUSES_PAD // N_TILES
N_CHUNKS = CLAUSES_PER_TILE // L


def _sigmoid(x):
    return 1.0 / (1.0 + jnp.exp(-x))


def _sat_body(logits_hbm, vars_hbm, signs_hbm,
              assign_hbm, sats_hbm, min_hbm, cnt_hbm,
              table_v, vars_v, signs_v, sat_v, sig_v, min_v, cnt_v):
    c = lax.axis_index("c")
    s = lax.axis_index("s")
    wid = c * 16 + s

    pltpu.sync_copy(logits_hbm, table_v)

    cbase = wid * CLAUSES_PER_TILE
    for j in range(3):
        pltpu.sync_copy(
            vars_hbm.at[pl.ds(j * CLAUSES_PAD + cbase, CLAUSES_PER_TILE)],
            vars_v.at[pl.ds(j * CLAUSES_PER_TILE, CLAUSES_PER_TILE)])
        pltpu.sync_copy(
            signs_hbm.at[pl.ds(j * CLAUSES_PAD + cbase, CLAUSES_PER_TILE)],
            signs_v.at[pl.ds(j * CLAUSES_PER_TILE, CLAUSES_PER_TILE)])

    vbase = wid * VARS_PER_TILE

    def sig_body(i, carry):
        x = table_v[pl.ds(vbase + i * L, L)]
        sig_v[pl.ds(i * L, L)] = _sigmoid(x)
        return carry

    lax.fori_loop(0, VARS_PER_TILE // L, sig_body, 0)
    pltpu.sync_copy(sig_v, assign_hbm.at[pl.ds(vbase, VARS_PER_TILE)])

    lanes = lax.iota(jnp.int32, L)

    def chunk_body(k, carry):
        mn, ct = carry
        col = k * L
        m = None
        for j in range(3):
            idx = vars_v[pl.ds(j * CLAUSES_PER_TILE + col, L)]
            x = plsc.load_gather(table_v, [idx])
            sg = signs_v[pl.ds(j * CLAUSES_PER_TILE + col, L)]
            lit = jnp.where(sg == 1, x, -x)
            m = lit if m is None else jnp.maximum(m, lit)
        sat = _sigmoid(m)
        sat_v[pl.ds(col, L)] = sat
        valid = (cbase + col + lanes) < N_CLAUSES
        mn = jnp.minimum(mn, jnp.where(valid, sat, jnp.float32(2.0)))
        ct = ct + jnp.where(valid & (sat > 0.5),
                            jnp.float32(1.0), jnp.float32(0.0))
        return (mn, ct)

    mn0 = jnp.full((L,), 2.0, jnp.float32)
    ct0 = jnp.zeros((L,), jnp.float32)
    mn, ct = lax.fori_loop(0, N_CHUNKS, chunk_body, (mn0, ct0))

    pltpu.sync_copy(sat_v, sats_hbm.at[pl.ds(cbase, CLAUSES_PER_TILE)])
    min_v[...] = mn
    cnt_v[...] = ct
    pltpu.sync_copy(min_v, min_hbm.at[pl.ds(wid * L, L)])
    pltpu.sync_copy(cnt_v, cnt_hbm.at[pl.ds(wid * L, L)])


_sat_call = pl.kernel(
    _sat_body,
    out_type=[
        jax.ShapeDtypeStruct((VARS_PAD,), jnp.float32),
        jax.ShapeDtypeStruct((CLAUSES_PAD,), jnp.float32),
        jax.ShapeDtypeStruct((N_TILES * L,), jnp.float32),
        jax.ShapeDtypeStruct((N_TILES * L,), jnp.float32),
    ],
    mesh=plsc.VectorSubcoreMesh(core_axis_name="c", subcore_axis_name="s"),
    compiler_params=pltpu.CompilerParams(needs_layout_passes=False),
    scratch_types=[
        pltpu.VMEM((VARS_PAD,), jnp.float32),
        pltpu.VMEM((3 * CLAUSES_PER_TILE,), jnp.int32),
        pltpu.VMEM((3 * CLAUSES_PER_TILE,), jnp.int32),
        pltpu.VMEM((CLAUSES_PER_TILE,), jnp.float32),
        pltpu.VMEM((VARS_PER_TILE,), jnp.float32),
        pltpu.VMEM((L,), jnp.float32),
        pltpu.VMEM((L,), jnp.float32),
    ],
)


@jax.jit
def kernel(assignment_logits, clause_vars, clause_signs):
    logits_p = jnp.pad(assignment_logits, (0, VARS_PAD - N_VARS))
    vars_t = jnp.pad(clause_vars.astype(jnp.int32),
                     ((0, CLAUSES_PAD - N_CLAUSES), (0, 0))).T.reshape(-1)
    signs_t = jnp.pad(clause_signs.astype(jnp.int32),
                      ((0, CLAUSES_PAD - N_CLAUSES), (0, 0))).T.reshape(-1)
    assign_p, sats_p, mins, cnts = _sat_call(logits_p, vars_t, signs_t)
    assignments = assign_p[:N_VARS]
    clause_satisfactions = sats_p[:N_CLAUSES]
    all_satisfied = jnp.min(mins)
    n_satisfied = jnp.sum(cnts)
    return (assignments, clause_satisfactions, all_satisfied, n_satisfied)
